# baseline (device time: 263940 ns/iter reference)
import numpy as np
import jax
import jax.numpy as jnp
from jax import lax
from jax.experimental import pallas as pl
from jax.experimental.pallas import tpu as pltpu

N_DEV = 32
M_CH = 128
N_SUB = 512
N_CHAN = 4

_LOG_ORDER = []
for _z in range(4):
    for _yi, _y in enumerate(range(4)):
        _row = [(0, _y, _z), (1, _y, _z)]
        if _yi % 2:
            _row.reverse()
        _LOG_ORDER.extend(_row)

_P = [(0, 0), (1, 0), (2, 0), (3, 0), (3, 1), (3, 2), (3, 3), (2, 3),
      (2, 2), (2, 1), (1, 1), (1, 2), (1, 3), (0, 3), (0, 2), (0, 1)]
_HAM = [(0, y, z) for y, z in _P] + [(1, y, z) for y, z in reversed(_P)]

_POS_OF_LOG = [_HAM.index(c) for c in _LOG_ORDER]
_RING_LOG = [_LOG_ORDER.index(c) for c in _HAM]

_NXT = np.array([_RING_LOG[(_POS_OF_LOG[d] + 1) % N_DEV] for d in range(N_DEV)],
                np.int32)
_PRV = np.array([_RING_LOG[(_POS_OF_LOG[d] - 1) % N_DEV] for d in range(N_DEV)],
                np.int32)
_CFWD = np.array([[_RING_LOG[(_POS_OF_LOG[d] - 1 - s) % N_DEV]
                   for s in range(N_DEV - 1)] for d in range(N_DEV)], np.int32)
_CREV = np.array([[_RING_LOG[(_POS_OF_LOG[d] + 1 + s) % N_DEV]
                   for s in range(N_DEV - 1)] for d in range(N_DEV)], np.int32)


def kernel(x, w_mat):
    m_total, k_shard = x.shape
    _, n = w_mat.shape

    my = lax.axis_index("i")
    nxt = jnp.take(jnp.asarray(_NXT), my).reshape(1)
    prv = jnp.take(jnp.asarray(_PRV), my).reshape(1)
    cfwd = lax.dynamic_index_in_dim(jnp.asarray(_CFWD), my, 0, keepdims=False)
    crev = lax.dynamic_index_in_dim(jnp.asarray(_CREV), my, 0, keepdims=False)

    def body(nxt_ref, prv_ref, cfwd_ref, crev_ref, x_ref, w_ref, out_ref,
             comm, pbuf, *sems):
        send_sems = sems[0:N_CHAN]
        recv_sems = sems[N_CHAN:2 * N_CHAN]
        credits = sems[2 * N_CHAN:3 * N_CHAN]

        my_id = lax.axis_index("i")
        nxt_id = nxt_ref[0]
        prv_id = prv_ref[0]
        dest = [nxt_id, nxt_id, prv_id, prv_id]
        upstream = [prv_id, prv_id, nxt_id, nxt_id]

        def part(s, ch):
            c = cfwd_ref[s] if ch < 2 else crev_ref[s]
            return jnp.dot(
                x_ref[pl.ds(c * M_CH, M_CH), :],
                w_ref[:, ch * N_SUB:(ch + 1) * N_SUB],
                preferred_element_type=jnp.float32,
            )

        def mk(s, ch):
            return pltpu.make_async_remote_copy(
                src_ref=comm.at[ch, s % 2],
                dst_ref=comm.at[ch, (s + 1) % 2],
                send_sem=send_sems[ch].at[s % 2],
                recv_sem=recv_sems[ch].at[(s + 1) % 2],
                device_id=(dest[ch],),
                device_id_type=pl.DeviceIdType.MESH,
            )

        for ch in range(N_CHAN):
            comm[ch, 0] = part(0, ch)
        barrier_sem = pltpu.get_barrier_semaphore()
        pl.semaphore_signal(
            barrier_sem, inc=1, device_id=(nxt_id,),
            device_id_type=pl.DeviceIdType.MESH,
        )
        pl.semaphore_signal(
            barrier_sem, inc=1, device_id=(prv_id,),
            device_id_type=pl.DeviceIdType.MESH,
        )
        pl.semaphore_wait(barrier_sem, 2)
        for ch in range(N_CHAN):
            mk(0, ch).start()

        for s in range(1, N_DEV - 1):
            slot = s % 2
            for ch in range(N_CHAN):
                pbuf[ch] = part(s, ch)
            for ch in range(N_CHAN):
                mk(s - 1, ch).wait_recv()
                comm[ch, slot] = comm[ch, slot] + pbuf[ch]
            for ch in range(N_CHAN):
                mk(s - 1, ch).wait_send()
                pl.semaphore_signal(
                    credits[ch], inc=1, device_id=(upstream[ch],),
                    device_id_type=pl.DeviceIdType.MESH,
                )
            for ch in range(N_CHAN):
                pl.semaphore_wait(credits[ch], 1)
                mk(s, ch).start()

        for ch in range(N_CHAN):
            pbuf[ch] = jnp.dot(
                x_ref[pl.ds(my_id * M_CH, M_CH), :],
                w_ref[:, ch * N_SUB:(ch + 1) * N_SUB],
                preferred_element_type=jnp.float32,
            )
        for ch in range(N_CHAN):
            mk(N_DEV - 2, ch).wait_recv()
            out_ref[:, ch * N_SUB:(ch + 1) * N_SUB] = comm[ch, 1] + pbuf[ch]
        for ch in range(N_CHAN):
            mk(N_DEV - 2, ch).wait_send()

    return pl.pallas_call(
        body,
        out_shape=jax.ShapeDtypeStruct((M_CH, n), jnp.float32),
        in_specs=[
            pl.BlockSpec(memory_space=pltpu.SMEM),
            pl.BlockSpec(memory_space=pltpu.SMEM),
            pl.BlockSpec(memory_space=pltpu.SMEM),
            pl.BlockSpec(memory_space=pltpu.SMEM),
            pl.BlockSpec(memory_space=pltpu.VMEM),
            pl.BlockSpec(memory_space=pltpu.VMEM),
        ],
        out_specs=pl.BlockSpec(memory_space=pltpu.VMEM),
        scratch_shapes=[
            pltpu.VMEM((N_CHAN, 2, M_CH, N_SUB), jnp.float32),
            pltpu.VMEM((N_CHAN, M_CH, N_SUB), jnp.float32),
        ]
        + [pltpu.SemaphoreType.DMA((2,)) for _ in range(N_CHAN)]
        + [pltpu.SemaphoreType.DMA((2,)) for _ in range(N_CHAN)]
        + [pltpu.SemaphoreType.REGULAR for _ in range(N_CHAN)],
        compiler_params=pltpu.CompilerParams(collective_id=0),
    )(nxt, prv, cfwd, crev, x, w_mat)
